# baseline (speedup 1.0000x reference)
import functools

import jax
import jax.numpy as jnp
from jax import lax
from jax.experimental import pallas as pl
from jax.experimental.pallas import tpu as pltpu
from jax.experimental.pallas import tpu_sc as plsc

NA = 16384
NJ = 50
D = 64
NW = 32
APW = NA // NW
RCH = 8
NCH = APW // RCH
NRING = 4
LOOK = NRING - 1
assert NCH % NRING == 0

_mesh = plsc.VectorSubcoreMesh(core_axis_name="c", subcore_axis_name="s")


@functools.partial(
    pl.kernel,
    mesh=_mesh,
    out_type=jax.ShapeDtypeStruct((NA, NJ, D), jnp.float32),
    scratch_types=[
        pltpu.VMEM((APW, NJ), jnp.int32),
        pltpu.VMEM((NRING, RCH, NJ, D), jnp.float32),
        [pltpu.SemaphoreType.DMA] * NRING,
        [pltpu.SemaphoreType.DMA] * NRING,
    ],
    compiler_params=pltpu.CompilerParams(use_tc_tiling_on_sc=False),
)
def _gather_kernel(idx_hbm, tab_hbm, out_hbm, idx_v, rows_v, gsems, osems):
    wid = lax.axis_index("s") * 2 + lax.axis_index("c")
    base = wid * APW
    pltpu.sync_copy(idx_hbm.at[pl.ds(base, APW)], idx_v)

    def fire(c, k):
        for j in range(RCH):
            pltpu.async_copy(
                tab_hbm.at[idx_v.at[c * RCH + j]],
                rows_v.at[k].at[j],
                gsems[k],
            )

    def drain(c, k):
        for j in range(RCH):
            pltpu.make_async_copy(
                tab_hbm.at[idx_v.at[c * RCH + j]],
                rows_v.at[k].at[j],
                gsems[k],
            ).wait()

    def out_desc(c, k):
        return pltpu.make_async_copy(
            rows_v.at[k], out_hbm.at[pl.ds(base + c * RCH, RCH)], osems[k]
        )

    for c in range(LOOK):
        fire(c, c)

    def body(i, _):
        c0 = NRING * i
        for k in range(NRING):
            c = c0 + k
            nxt_k = (k + LOOK) % NRING
            if k == 0:
                @pl.when(i > 0)
                def _():
                    out_desc(c - 1, nxt_k).wait()
            else:
                out_desc(c - 1, nxt_k).wait()

            @pl.when(c + LOOK < NCH)
            def _():
                fire(c + LOOK, nxt_k)

            drain(c, k)
            out_desc(c, k).start()
        return 0

    lax.fori_loop(0, NCH // NRING, body, 0)
    out_desc(NCH - 1, (NCH - 1) % NRING).wait()


def kernel(action, action_embeddings):
    return _gather_kernel(action.astype(jnp.int32), action_embeddings)
